# baseline (speedup 1.0000x reference)
import functools

import jax
import jax.numpy as jnp
from jax import lax
from jax.experimental import pallas as pl
from jax.experimental.pallas import tpu as pltpu
from jax.experimental.pallas import tpu_sc as plsc

N = 10000
E = 320000
D = 128
H = 128
C = 40

NC = 2
NS = 16
NW = NC * NS

NP = 10240
EPW = 10240
EPAD = NW * EPW
CHUNK = 128
NCHUNKS = EPW // CHUNK
RPT = NP // NS
TRASH = N
NTRASH = NP - N


def _zero_vmem_2d(ref, nrows, ncols):
    zv = jnp.zeros((16,), jnp.float32)

    def row(i, _):
        for j in range(ncols // 16):
            ref[i, pl.ds(j * 16, 16)] = zv
        return 0

    lax.fori_loop(0, nrows, row, 0, unroll=False)


_MESH = plsc.VectorSubcoreMesh(core_axis_name="c", subcore_axis_name="s")


CFIRE = 8


@functools.partial(
    pl.kernel,
    mesh=_MESH,
    out_type=jax.ShapeDtypeStruct((NC, NP), jnp.float32),
    scratch_types=[
        pltpu.VMEM((NCHUNKS, CHUNK), jnp.int32),
        pltpu.VMEM((CHUNK,), jnp.float32),
        pltpu.VMEM((RPT,), jnp.float32),
        pltpu.VMEM_SHARED((NP,), jnp.float32),
        pltpu.SemaphoreType.DMA,
        pltpu.SemaphoreType.DMA,
    ],
)
def _sc_count(dst_hbm, out_hbm, didx2d, ones_v, zrow, cnt_sh, isem, csem):
    c = lax.axis_index("c")
    s = lax.axis_index("s")
    wid = s * NC + c

    hidx = pltpu.async_copy(dst_hbm.at[wid], didx2d, isem)

    one = jnp.ones((16,), jnp.float32)
    zv = jnp.zeros((16,), jnp.float32)
    for j in range(CHUNK // 16):
        ones_v[pl.ds(j * 16, 16)] = one

    def zrow_body(i, _):
        zrow[pl.ds(i * 16, 16)] = zv
        return 0

    lax.fori_loop(0, RPT // 16, zrow_body, 0, unroll=False)
    pltpu.sync_copy(zrow, cnt_sh.at[pl.ds(s * RPT, RPT)])
    hidx.wait()
    plsc.subcore_barrier()

    def step(b, _):
        for j in range(CFIRE):
            pltpu.async_copy(
                ones_v, cnt_sh.at[didx2d.at[b * CFIRE + j]], csem, add=True
            )
        for j in range(CFIRE):
            pltpu.make_async_copy(ones_v, cnt_sh.at[didx2d.at[0]], csem).wait()
        return 0

    lax.fori_loop(0, NCHUNKS // CFIRE, step, 0, unroll=False)
    plsc.subcore_barrier()
    pltpu.sync_copy(
        cnt_sh.at[pl.ds(s * RPT, RPT)], out_hbm.at[c].at[pl.ds(s * RPT, RPT)]
    )


IBLK = 16
NBLK = NCHUNKS // IBLK


@functools.partial(
    pl.kernel,
    mesh=_MESH,
    out_type=jax.ShapeDtypeStruct((NC, NP, H), jnp.float32),
    scratch_types=[
        pltpu.VMEM((2, IBLK, CHUNK), jnp.int32),
        pltpu.VMEM((2, IBLK, CHUNK), jnp.int32),
        pltpu.VMEM((2, CHUNK, H), jnp.float32),
        pltpu.VMEM_SHARED((NP, H), jnp.float32),
        pltpu.SemaphoreType.DMA,
        pltpu.SemaphoreType.DMA,
        pltpu.SemaphoreType.DMA,
        pltpu.SemaphoreType.DMA,
        pltpu.SemaphoreType.DMA,
    ],
)
def _sc_scatter(y_hbm, src_hbm, dst_hbm, out_hbm, si, di, rows, acc_sh,
                gsem0, gsem1, ssem0, ssem1, isem):
    c = lax.axis_index("c")
    s = lax.axis_index("s")
    wid = s * NC + c

    hs = pltpu.async_copy(src_hbm.at[wid].at[pl.ds(0, IBLK)], si.at[0], isem)
    hd = pltpu.async_copy(dst_hbm.at[wid].at[pl.ds(0, IBLK)], di.at[0], isem)
    _zero_vmem_2d(rows.at[0], CHUNK, H)
    for k in range(RPT // CHUNK):
        pltpu.sync_copy(rows.at[0], acc_sh.at[pl.ds(s * RPT + k * CHUNK, CHUNK)])
    hs.wait()
    hd.wait()
    plsc.subcore_barrier()

    for b in range(NBLK):
        p = b % 2
        q = (b + 1) % 2
        sb = si.at[p]
        db = di.at[p]
        if b + 1 < NBLK:
            nxt = pl.ds((b + 1) * IBLK, IBLK)
            hs = pltpu.async_copy(src_hbm.at[wid].at[nxt], si.at[q], isem)
            hd = pltpu.async_copy(dst_hbm.at[wid].at[nxt], di.at[q], isem)
        pltpu.async_copy(y_hbm.at[sb.at[0]], rows.at[0], gsem0)
        pltpu.async_copy(y_hbm.at[sb.at[1]], rows.at[1], gsem1)

        def step(g, _, sb=sb, db=db):
            c0 = 2 * g
            c1 = c0 + 1
            pltpu.make_async_copy(y_hbm.at[sb.at[c0]], rows.at[0], gsem0).wait()
            pltpu.async_copy(rows.at[0], acc_sh.at[db.at[c0]], ssem0, add=True)
            pltpu.make_async_copy(y_hbm.at[sb.at[c1]], rows.at[1], gsem1).wait()
            pltpu.async_copy(rows.at[1], acc_sh.at[db.at[c1]], ssem1, add=True)
            pltpu.make_async_copy(rows.at[0], acc_sh.at[db.at[c0]], ssem0).wait()
            pltpu.async_copy(y_hbm.at[sb.at[c0 + 2]], rows.at[0], gsem0)
            pltpu.make_async_copy(rows.at[1], acc_sh.at[db.at[c1]], ssem1).wait()
            pltpu.async_copy(y_hbm.at[sb.at[c1 + 2]], rows.at[1], gsem1)
            return 0

        lax.fori_loop(0, IBLK // 2 - 1, step, 0, unroll=False)
        k0 = IBLK - 2
        pltpu.make_async_copy(y_hbm.at[sb.at[k0]], rows.at[0], gsem0).wait()
        pltpu.async_copy(rows.at[0], acc_sh.at[db.at[k0]], ssem0, add=True)
        pltpu.make_async_copy(y_hbm.at[sb.at[k0 + 1]], rows.at[1], gsem1).wait()
        pltpu.async_copy(rows.at[1], acc_sh.at[db.at[k0 + 1]], ssem1, add=True)
        pltpu.make_async_copy(rows.at[0], acc_sh.at[db.at[k0]], ssem0).wait()
        pltpu.make_async_copy(rows.at[1], acc_sh.at[db.at[k0 + 1]], ssem1).wait()
        if b + 1 < NBLK:
            hs.wait()
            hd.wait()

    plsc.subcore_barrier()
    for k in range(RPT // CHUNK):
        r0 = s * RPT + k * CHUNK
        pltpu.sync_copy(
            acc_sh.at[pl.ds(r0, CHUNK)], out_hbm.at[c].at[pl.ds(r0, CHUNK)]
        )


def _elu(v):
    return jnp.where(v > 0, v, jnp.exp(jnp.minimum(v, 0.0)) - 1.0)


def _tc_dense1(x, W1, cnt):
    def body(x_ref, w_ref, cnt_ref, y_ref, dinv_ref):
        cnt = cnt_ref[...]
        deg = cnt[0, :N] + cnt[1, :N] + 1.0
        dinv = jnp.reshape(lax.rsqrt(deg), (N, 1))
        xw = jnp.dot(x_ref[...], w_ref[...], preferred_element_type=jnp.float32)
        y_ref[...] = xw * dinv
        dinv_ref[...] = dinv

    return pl.pallas_call(
        body,
        out_shape=(
            jax.ShapeDtypeStruct((N, H), jnp.float32),
            jax.ShapeDtypeStruct((N, 1), jnp.float32),
        ),
    )(x, W1, cnt)


def _tc_dense2(acc, y1, dinv, b1, W2):
    def body(acc_ref, y1_ref, dinv_ref, b1_ref, w_ref, y2_ref):
        dinv = dinv_ref[...]
        agg = acc_ref[0, :N, :] + acc_ref[1, :N, :] + y1_ref[...]
        h = _elu(dinv * agg + b1_ref[...])
        y2_ref[...] = (
            jnp.dot(h, w_ref[...], preferred_element_type=jnp.float32) * dinv
        )

    return pl.pallas_call(
        body,
        out_shape=jax.ShapeDtypeStruct((N, H), jnp.float32),
    )(acc, y1, dinv, b1, W2)


def _tc_dense3(acc, y2, dinv, b2, Wl, bl):
    def body(acc_ref, y2_ref, dinv_ref, b2_ref, wl_ref, bl_ref, out_ref):
        dinv = dinv_ref[...]
        agg = acc_ref[0, :N, :] + acc_ref[1, :N, :] + y2_ref[...]
        h = _elu(dinv * agg + b2_ref[...])
        logits = (
            jnp.dot(h, wl_ref[...], preferred_element_type=jnp.float32)
            + bl_ref[...]
        )
        m = jnp.max(logits, axis=1, keepdims=True)
        lse = m + jnp.log(jnp.sum(jnp.exp(logits - m), axis=1, keepdims=True))
        out_ref[...] = logits - lse

    return pl.pallas_call(
        body,
        out_shape=jax.ShapeDtypeStruct((N, C), jnp.float32),
    )(acc, y2, dinv, b2, Wl, bl)


def kernel(x, edge_index, W1, b1, W2, b2, Wl, bl):
    ei = edge_index.astype(jnp.int32)
    npad = EPAD - E
    pad_src = jnp.arange(npad, dtype=jnp.int32) % N
    pad_dst = TRASH + jnp.arange(npad, dtype=jnp.int32) % NTRASH
    src = jnp.reshape(jnp.concatenate([ei[0], pad_src]), (NW, NCHUNKS, CHUNK))
    dst = jnp.reshape(jnp.concatenate([ei[1], pad_dst]), (NW, NCHUNKS, CHUNK))

    cnt = _sc_count(dst)
    y1, dinv = _tc_dense1(x, W1, cnt)
    acc1 = _sc_scatter(y1, src, dst)
    y2 = _tc_dense2(acc1, y1, dinv, jnp.reshape(b1, (1, H)), W2)
    acc2 = _sc_scatter(y2, src, dst)
    return _tc_dense3(
        acc2, y2, dinv, jnp.reshape(b2, (1, H)), Wl, jnp.reshape(bl, (1, C))
    )


# speedup vs baseline: 1.2317x; 1.2317x over previous
import functools

import jax
import jax.numpy as jnp
from jax import lax
from jax.experimental import pallas as pl
from jax.experimental.pallas import tpu as pltpu
from jax.experimental.pallas import tpu_sc as plsc

N = 10000
E = 320000
D = 128
H = 128
C = 40

NC = 2
NS = 16
NW = NC * NS

NP = 10240
EPW = E // NW
CHUNK = 125
NCHUNKS = EPW // CHUNK
RPT = NP // NS
ZCH = 64


def _zero_vmem_2d(ref, nrows, ncols):
    zv = jnp.zeros((16,), jnp.float32)

    def row(i, _):
        for j in range(ncols // 16):
            ref[i, pl.ds(j * 16, 16)] = zv
        return 0

    lax.fori_loop(0, nrows, row, 0, unroll=False)


_MESH = plsc.VectorSubcoreMesh(core_axis_name="c", subcore_axis_name="s")


CFIRE = 8


@functools.partial(
    pl.kernel,
    mesh=_MESH,
    out_type=jax.ShapeDtypeStruct((NC, NP), jnp.float32),
    scratch_types=[
        pltpu.VMEM((NCHUNKS, CHUNK), jnp.int32),
        pltpu.VMEM((ZCH,), jnp.float32),
        pltpu.VMEM((RPT,), jnp.float32),
        pltpu.VMEM_SHARED((NP,), jnp.float32),
        pltpu.SemaphoreType.DMA,
        pltpu.SemaphoreType.DMA,
    ],
)
def _sc_count(dst_hbm, out_hbm, didx2d, ones_v, zrow, cnt_sh, isem, csem):
    c = lax.axis_index("c")
    s = lax.axis_index("s")
    wid = s * NC + c

    hidx = pltpu.async_copy(dst_hbm.at[wid], didx2d, isem)

    one = jnp.ones((16,), jnp.float32)
    zv = jnp.zeros((16,), jnp.float32)
    for j in range(ZCH // 16):
        ones_v[pl.ds(j * 16, 16)] = one

    def zrow_body(i, _):
        zrow[pl.ds(i * 16, 16)] = zv
        return 0

    lax.fori_loop(0, RPT // 16, zrow_body, 0, unroll=False)
    pltpu.sync_copy(zrow, cnt_sh.at[pl.ds(s * RPT, RPT)])
    hidx.wait()
    plsc.subcore_barrier()

    def step(b, _):
        for j in range(CFIRE):
            pltpu.async_copy(
                ones_v.at[pl.ds(0, CHUNK)], cnt_sh.at[didx2d.at[b * CFIRE + j]],
                csem, add=True,
            )
        for j in range(CFIRE):
            pltpu.make_async_copy(
                ones_v.at[pl.ds(0, CHUNK)], cnt_sh.at[didx2d.at[0]], csem
            ).wait()
        return 0

    lax.fori_loop(0, NCHUNKS // CFIRE, step, 0, unroll=False)
    plsc.subcore_barrier()
    pltpu.sync_copy(
        cnt_sh.at[pl.ds(s * RPT, RPT)], out_hbm.at[c].at[pl.ds(s * RPT, RPT)]
    )


IBLK = 16
NBLK = NCHUNKS // IBLK


@functools.partial(
    pl.kernel,
    mesh=_MESH,
    out_type=jax.ShapeDtypeStruct((NC, NP, H), jnp.float32),
    scratch_types=[
        pltpu.VMEM((2, IBLK, CHUNK), jnp.int32),
        pltpu.VMEM((2, IBLK, CHUNK), jnp.int32),
        pltpu.VMEM((2, CHUNK, H), jnp.float32),
        pltpu.VMEM((ZCH, H), jnp.float32),
        pltpu.VMEM_SHARED((NP, H), jnp.float32),
        pltpu.SemaphoreType.DMA,
        pltpu.SemaphoreType.DMA,
        pltpu.SemaphoreType.DMA,
    ],
)
def _sc_scatter(y_hbm, src_hbm, dst_hbm, out_hbm, si, di, rows, zrow, acc_sh,
                gsem0, gsem1, isem):
    c = lax.axis_index("c")
    s = lax.axis_index("s")
    wid = s * NC + c

    hs = pltpu.async_copy(src_hbm.at[wid].at[pl.ds(0, IBLK)], si.at[0], isem)
    hd = pltpu.async_copy(dst_hbm.at[wid].at[pl.ds(0, IBLK)], di.at[0], isem)
    _zero_vmem_2d(zrow, ZCH, H)
    hs.wait()
    hd.wait()
    pltpu.async_copy(y_hbm.at[si.at[0].at[0]], rows.at[0], gsem0)
    pltpu.async_copy(y_hbm.at[si.at[0].at[1]], rows.at[1], gsem1)
    for k in range(RPT // ZCH):
        pltpu.sync_copy(zrow, acc_sh.at[pl.ds(s * RPT + k * ZCH, ZCH)])
    plsc.subcore_barrier()

    for b in range(NBLK):
        p = b % 2
        q = (b + 1) % 2
        sb = si.at[p]
        db = di.at[p]
        if b + 1 < NBLK:
            nxt = pl.ds((b + 1) * IBLK, IBLK)
            hs = pltpu.async_copy(src_hbm.at[wid].at[nxt], si.at[q], isem)
            hd = pltpu.async_copy(dst_hbm.at[wid].at[nxt], di.at[q], isem)
        if b > 0:
            pltpu.async_copy(y_hbm.at[sb.at[0]], rows.at[0], gsem0)
            pltpu.async_copy(y_hbm.at[sb.at[1]], rows.at[1], gsem1)

        def step(g, _, sb=sb, db=db):
            c0 = 2 * g
            pltpu.make_async_copy(y_hbm.at[sb.at[c0]], rows.at[0], gsem0).wait()
            pltpu.sync_copy(rows.at[0], acc_sh.at[db.at[c0]], add=True)
            pltpu.async_copy(y_hbm.at[sb.at[c0 + 2]], rows.at[0], gsem0)
            c1 = c0 + 1
            pltpu.make_async_copy(y_hbm.at[sb.at[c1]], rows.at[1], gsem1).wait()
            pltpu.sync_copy(rows.at[1], acc_sh.at[db.at[c1]], add=True)
            pltpu.async_copy(y_hbm.at[sb.at[c1 + 2]], rows.at[1], gsem1)
            return 0

        lax.fori_loop(0, IBLK // 2 - 1, step, 0, unroll=False)
        k0 = IBLK - 2
        pltpu.make_async_copy(y_hbm.at[sb.at[k0]], rows.at[0], gsem0).wait()
        pltpu.sync_copy(rows.at[0], acc_sh.at[db.at[k0]], add=True)
        pltpu.make_async_copy(y_hbm.at[sb.at[k0 + 1]], rows.at[1], gsem1).wait()
        pltpu.sync_copy(rows.at[1], acc_sh.at[db.at[k0 + 1]], add=True)
        if b + 1 < NBLK:
            hs.wait()
            hd.wait()

    plsc.subcore_barrier()
    for k in range(RPT // ZCH):
        r0 = s * RPT + k * ZCH
        pltpu.sync_copy(
            acc_sh.at[pl.ds(r0, ZCH)], out_hbm.at[c].at[pl.ds(r0, ZCH)]
        )


def _elu(v):
    return jnp.where(v > 0, v, jnp.exp(jnp.minimum(v, 0.0)) - 1.0)


def _tc_dense1(x, W1, cnt):
    def body(x_ref, w_ref, cnt_ref, y_ref, dinv_ref):
        cnt = cnt_ref[...]
        deg = cnt[0, :N] + cnt[1, :N] + 1.0
        dinv = jnp.reshape(lax.rsqrt(deg), (N, 1))
        xw = jnp.dot(x_ref[...], w_ref[...], preferred_element_type=jnp.float32)
        y_ref[...] = xw * dinv
        dinv_ref[...] = dinv

    return pl.pallas_call(
        body,
        out_shape=(
            jax.ShapeDtypeStruct((N, H), jnp.float32),
            jax.ShapeDtypeStruct((N, 1), jnp.float32),
        ),
    )(x, W1, cnt)


def _tc_dense2(acc, y1, dinv, b1, W2):
    def body(acc_ref, y1_ref, dinv_ref, b1_ref, w_ref, y2_ref):
        dinv = dinv_ref[...]
        agg = acc_ref[0, :N, :] + acc_ref[1, :N, :] + y1_ref[...]
        h = _elu(dinv * agg + b1_ref[...])
        y2_ref[...] = (
            jnp.dot(h, w_ref[...], preferred_element_type=jnp.float32) * dinv
        )

    return pl.pallas_call(
        body,
        out_shape=jax.ShapeDtypeStruct((N, H), jnp.float32),
    )(acc, y1, dinv, b1, W2)


def _tc_dense3(acc, y2, dinv, b2, Wl, bl):
    def body(acc_ref, y2_ref, dinv_ref, b2_ref, wl_ref, bl_ref, out_ref):
        dinv = dinv_ref[...]
        agg = acc_ref[0, :N, :] + acc_ref[1, :N, :] + y2_ref[...]
        h = _elu(dinv * agg + b2_ref[...])
        logits = (
            jnp.dot(h, wl_ref[...], preferred_element_type=jnp.float32)
            + bl_ref[...]
        )
        m = jnp.max(logits, axis=1, keepdims=True)
        lse = m + jnp.log(jnp.sum(jnp.exp(logits - m), axis=1, keepdims=True))
        out_ref[...] = logits - lse

    return pl.pallas_call(
        body,
        out_shape=jax.ShapeDtypeStruct((N, C), jnp.float32),
    )(acc, y2, dinv, b2, Wl, bl)


def kernel(x, edge_index, W1, b1, W2, b2, Wl, bl):
    ei = edge_index.astype(jnp.int32)
    src = jnp.reshape(ei[0], (NW, NCHUNKS, CHUNK))
    dst = jnp.reshape(ei[1], (NW, NCHUNKS, CHUNK))

    cnt = _sc_count(dst)
    y1, dinv = _tc_dense1(x, W1, cnt)
    acc1 = _sc_scatter(y1, src, dst)
    y2 = _tc_dense2(acc1, y1, dinv, jnp.reshape(b1, (1, H)), W2)
    acc2 = _sc_scatter(y2, src, dst)
    return _tc_dense3(
        acc2, y2, dinv, jnp.reshape(b2, (1, H)), Wl, jnp.reshape(bl, (1, C))
    )


# speedup vs baseline: 1.2692x; 1.0305x over previous
import functools

import jax
import jax.numpy as jnp
from jax import lax
from jax.experimental import pallas as pl
from jax.experimental.pallas import tpu as pltpu
from jax.experimental.pallas import tpu_sc as plsc

N = 10000
E = 320000
D = 128
H = 128
C = 40

NC = 2
NS = 16
NW = NC * NS

NP = 10240
EPW = E // NW
CHUNK = 125
NCHUNKS = EPW // CHUNK
RPT = NP // NS
ZCH = 64


def _zero_vmem_2d(ref, nrows, ncols):
    zv = jnp.zeros((16,), jnp.float32)

    def row(i, _):
        for j in range(ncols // 16):
            ref[i, pl.ds(j * 16, 16)] = zv
        return 0

    lax.fori_loop(0, nrows, row, 0, unroll=False)


_MESH = plsc.VectorSubcoreMesh(core_axis_name="c", subcore_axis_name="s")


CFIRE = 8


@functools.partial(
    pl.kernel,
    mesh=_MESH,
    out_type=jax.ShapeDtypeStruct((NC, NP), jnp.float32),
    scratch_types=[
        pltpu.VMEM((NCHUNKS, CHUNK), jnp.int32),
        pltpu.VMEM((ZCH,), jnp.float32),
        pltpu.VMEM((RPT,), jnp.float32),
        pltpu.VMEM_SHARED((NP,), jnp.float32),
        pltpu.SemaphoreType.DMA,
        pltpu.SemaphoreType.DMA,
    ],
)
def _sc_count(e_hbm, out_hbm, didx2d, ones_v, zrow, cnt_sh, isem, csem):
    c = lax.axis_index("c")
    s = lax.axis_index("s")
    wid = s * NC + c

    hidx = pltpu.async_copy(e_hbm.at[1].at[wid], didx2d, isem)

    one = jnp.ones((16,), jnp.float32)
    zv = jnp.zeros((16,), jnp.float32)
    for j in range(ZCH // 16):
        ones_v[pl.ds(j * 16, 16)] = one

    def zrow_body(i, _):
        zrow[pl.ds(i * 16, 16)] = zv
        return 0

    lax.fori_loop(0, RPT // 16, zrow_body, 0, unroll=False)
    pltpu.sync_copy(zrow, cnt_sh.at[pl.ds(s * RPT, RPT)])
    hidx.wait()
    plsc.subcore_barrier()

    def step(b, _):
        for j in range(CFIRE):
            pltpu.async_copy(
                ones_v.at[pl.ds(0, CHUNK)], cnt_sh.at[didx2d.at[b * CFIRE + j]],
                csem, add=True,
            )
        for j in range(CFIRE):
            pltpu.make_async_copy(
                ones_v.at[pl.ds(0, CHUNK)], cnt_sh.at[didx2d.at[0]], csem
            ).wait()
        return 0

    lax.fori_loop(0, NCHUNKS // CFIRE, step, 0, unroll=False)
    plsc.subcore_barrier()
    pltpu.sync_copy(
        cnt_sh.at[pl.ds(s * RPT, RPT)], out_hbm.at[c].at[pl.ds(s * RPT, RPT)]
    )


IBLK = 16
NBLK = NCHUNKS // IBLK


@functools.partial(
    pl.kernel,
    mesh=_MESH,
    out_type=jax.ShapeDtypeStruct((NC, NP, H), jnp.float32),
    scratch_types=[
        pltpu.VMEM((2, IBLK, CHUNK), jnp.int32),
        pltpu.VMEM((2, IBLK, CHUNK), jnp.int32),
        pltpu.VMEM((2, CHUNK, H), jnp.float32),
        pltpu.VMEM((ZCH, H), jnp.float32),
        pltpu.VMEM_SHARED((NP, H), jnp.float32),
        pltpu.SemaphoreType.DMA,
        pltpu.SemaphoreType.DMA,
        pltpu.SemaphoreType.DMA,
    ],
)
def _sc_scatter(y_hbm, e_hbm, out_hbm, si, di, rows, zrow, acc_sh,
                gsem0, gsem1, isem):
    c = lax.axis_index("c")
    s = lax.axis_index("s")
    wid = s * NC + c

    hs = pltpu.async_copy(e_hbm.at[0].at[wid].at[pl.ds(0, IBLK)], si.at[0], isem)
    hd = pltpu.async_copy(e_hbm.at[1].at[wid].at[pl.ds(0, IBLK)], di.at[0], isem)
    _zero_vmem_2d(zrow, ZCH, H)
    hs.wait()
    hd.wait()
    pltpu.async_copy(y_hbm.at[si.at[0].at[0]], rows.at[0], gsem0)
    pltpu.async_copy(y_hbm.at[si.at[0].at[1]], rows.at[1], gsem1)
    for k in range(RPT // ZCH):
        pltpu.sync_copy(zrow, acc_sh.at[pl.ds(s * RPT + k * ZCH, ZCH)])
    plsc.subcore_barrier()

    for b in range(NBLK):
        p = b % 2
        q = (b + 1) % 2
        sb = si.at[p]
        db = di.at[p]
        if b + 1 < NBLK:
            nxt = pl.ds((b + 1) * IBLK, IBLK)
            hs = pltpu.async_copy(e_hbm.at[0].at[wid].at[nxt], si.at[q], isem)
            hd = pltpu.async_copy(e_hbm.at[1].at[wid].at[nxt], di.at[q], isem)
        if b > 0:
            pltpu.async_copy(y_hbm.at[sb.at[0]], rows.at[0], gsem0)
            pltpu.async_copy(y_hbm.at[sb.at[1]], rows.at[1], gsem1)

        def step(g, _, sb=sb, db=db):
            c0 = 2 * g
            pltpu.make_async_copy(y_hbm.at[sb.at[c0]], rows.at[0], gsem0).wait()
            pltpu.sync_copy(rows.at[0], acc_sh.at[db.at[c0]], add=True)
            pltpu.async_copy(y_hbm.at[sb.at[c0 + 2]], rows.at[0], gsem0)
            c1 = c0 + 1
            pltpu.make_async_copy(y_hbm.at[sb.at[c1]], rows.at[1], gsem1).wait()
            pltpu.sync_copy(rows.at[1], acc_sh.at[db.at[c1]], add=True)
            pltpu.async_copy(y_hbm.at[sb.at[c1 + 2]], rows.at[1], gsem1)
            return 0

        lax.fori_loop(0, IBLK // 2 - 1, step, 0, unroll=False)
        k0 = IBLK - 2
        pltpu.make_async_copy(y_hbm.at[sb.at[k0]], rows.at[0], gsem0).wait()
        pltpu.sync_copy(rows.at[0], acc_sh.at[db.at[k0]], add=True)
        pltpu.make_async_copy(y_hbm.at[sb.at[k0 + 1]], rows.at[1], gsem1).wait()
        pltpu.sync_copy(rows.at[1], acc_sh.at[db.at[k0 + 1]], add=True)
        if b + 1 < NBLK:
            hs.wait()
            hd.wait()

    plsc.subcore_barrier()
    for k in range(RPT // ZCH):
        r0 = s * RPT + k * ZCH
        pltpu.sync_copy(
            acc_sh.at[pl.ds(r0, ZCH)], out_hbm.at[c].at[pl.ds(r0, ZCH)]
        )


def _elu(v):
    return jnp.where(v > 0, v, jnp.exp(jnp.minimum(v, 0.0)) - 1.0)


def _tc_dense1(x, W1, cnt):
    def body(x_ref, w_ref, cnt_ref, y_ref, dinv_ref):
        cnt = cnt_ref[...]
        deg = cnt[0, :N] + cnt[1, :N] + 1.0
        dinv = jnp.reshape(lax.rsqrt(deg), (N, 1))
        xw = jnp.dot(x_ref[...], w_ref[...], preferred_element_type=jnp.float32)
        y_ref[...] = xw * dinv
        dinv_ref[...] = dinv

    return pl.pallas_call(
        body,
        out_shape=(
            jax.ShapeDtypeStruct((N, H), jnp.float32),
            jax.ShapeDtypeStruct((N, 1), jnp.float32),
        ),
    )(x, W1, cnt)


def _tc_dense2(acc, y1, dinv, b1, W2):
    def body(acc_ref, y1_ref, dinv_ref, b1_ref, w_ref, y2_ref):
        dinv = dinv_ref[...]
        agg = acc_ref[0, :N, :] + acc_ref[1, :N, :] + y1_ref[...]
        h = _elu(dinv * agg + b1_ref[...])
        y2_ref[...] = (
            jnp.dot(h, w_ref[...], preferred_element_type=jnp.float32) * dinv
        )

    return pl.pallas_call(
        body,
        out_shape=jax.ShapeDtypeStruct((N, H), jnp.float32),
    )(acc, y1, dinv, b1, W2)


def _tc_dense3(acc, y2, dinv, b2, Wl, bl):
    def body(acc_ref, y2_ref, dinv_ref, b2_ref, wl_ref, bl_ref, out_ref):
        dinv = dinv_ref[...]
        agg = acc_ref[0, :N, :] + acc_ref[1, :N, :] + y2_ref[...]
        h = _elu(dinv * agg + b2_ref[...])
        logits = (
            jnp.dot(h, wl_ref[...], preferred_element_type=jnp.float32)
            + bl_ref[...]
        )
        m = jnp.max(logits, axis=1, keepdims=True)
        lse = m + jnp.log(jnp.sum(jnp.exp(logits - m), axis=1, keepdims=True))
        out_ref[...] = logits - lse

    return pl.pallas_call(
        body,
        out_shape=jax.ShapeDtypeStruct((N, C), jnp.float32),
    )(acc, y2, dinv, b2, Wl, bl)


def kernel(x, edge_index, W1, b1, W2, b2, Wl, bl):
    e4 = jnp.reshape(edge_index.astype(jnp.int32), (2, NW, NCHUNKS, CHUNK))

    cnt = _sc_count(e4)
    y1, dinv = _tc_dense1(x, W1, cnt)
    acc1 = _sc_scatter(y1, e4)
    y2 = _tc_dense2(acc1, y1, dinv, jnp.reshape(b1, (1, H)), W2)
    acc2 = _sc_scatter(y2, e4)
    return _tc_dense3(
        acc2, y2, dinv, jnp.reshape(b2, (1, H)), Wl, jnp.reshape(bl, (1, C))
    )


# speedup vs baseline: 1.3053x; 1.0285x over previous
import functools

import jax
import jax.numpy as jnp
from jax import lax
from jax.experimental import pallas as pl
from jax.experimental.pallas import tpu as pltpu
from jax.experimental.pallas import tpu_sc as plsc

N = 10000
E = 320000
D = 128
H = 128
C = 40

NC = 2
NS = 16
NW = NC * NS

NP = 10240
EPW = E // NW
CHUNK = 125
NCHUNKS = EPW // CHUNK
RPT = NP // NS
ZCH = 64


def _zero_vmem_2d(ref, nrows, ncols):
    zv = jnp.zeros((16,), jnp.float32)

    def row(i, _):
        for j in range(ncols // 16):
            ref[i, pl.ds(j * 16, 16)] = zv
        return 0

    lax.fori_loop(0, nrows, row, 0, unroll=False)


_MESH = plsc.VectorSubcoreMesh(core_axis_name="c", subcore_axis_name="s")


CFIRE = 8


@functools.partial(
    pl.kernel,
    mesh=_MESH,
    out_type=jax.ShapeDtypeStruct((NC, NP), jnp.float32),
    scratch_types=[
        pltpu.VMEM((NCHUNKS, CHUNK), jnp.int32),
        pltpu.VMEM((ZCH,), jnp.float32),
        pltpu.VMEM((RPT,), jnp.float32),
        pltpu.VMEM_SHARED((NP,), jnp.float32),
        pltpu.SemaphoreType.DMA,
        pltpu.SemaphoreType.DMA,
    ],
)
def _sc_count(e_hbm, out_hbm, didx2d, ones_v, zrow, cnt_sh, isem, csem):
    c = lax.axis_index("c")
    s = lax.axis_index("s")
    wid = s * NC + c

    hidx = pltpu.async_copy(e_hbm.at[1].at[wid], didx2d, isem)

    one = jnp.ones((16,), jnp.float32)
    zv = jnp.zeros((16,), jnp.float32)
    for j in range(ZCH // 16):
        ones_v[pl.ds(j * 16, 16)] = one

    def zrow_body(i, _):
        zrow[pl.ds(i * 16, 16)] = zv
        return 0

    lax.fori_loop(0, RPT // 16, zrow_body, 0, unroll=False)
    pltpu.sync_copy(zrow, cnt_sh.at[pl.ds(s * RPT, RPT)])
    hidx.wait()
    plsc.subcore_barrier()

    def step(b, _):
        for j in range(CFIRE):
            pltpu.async_copy(
                ones_v.at[pl.ds(0, CHUNK)], cnt_sh.at[didx2d.at[b * CFIRE + j]],
                csem, add=True,
            )
        for j in range(CFIRE):
            pltpu.make_async_copy(
                ones_v.at[pl.ds(0, CHUNK)], cnt_sh.at[didx2d.at[0]], csem
            ).wait()
        return 0

    lax.fori_loop(0, NCHUNKS // CFIRE, step, 0, unroll=False)
    plsc.subcore_barrier()
    pltpu.sync_copy(
        cnt_sh.at[pl.ds(s * RPT, RPT)], out_hbm.at[c].at[pl.ds(s * RPT, RPT)]
    )


IBLK = 16
NBLK = NCHUNKS // IBLK


@functools.partial(
    pl.kernel,
    mesh=_MESH,
    out_type=jax.ShapeDtypeStruct((NC, NP, H), jnp.float32),
    scratch_types=[
        pltpu.VMEM((2, IBLK, CHUNK), jnp.int32),
        pltpu.VMEM((2, IBLK, CHUNK), jnp.int32),
        pltpu.VMEM((2, CHUNK, H), jnp.float32),
        pltpu.VMEM((ZCH, H), jnp.float32),
        pltpu.VMEM_SHARED((NP, H), jnp.float32),
        pltpu.SemaphoreType.DMA,
        pltpu.SemaphoreType.DMA,
        pltpu.SemaphoreType.DMA,
    ],
)
def _sc_scatter(y_hbm, e_hbm, out_hbm, si, di, rows, zrow, acc_sh,
                gsem0, gsem1, isem):
    c = lax.axis_index("c")
    s = lax.axis_index("s")
    wid = s * NC + c

    hs = pltpu.async_copy(e_hbm.at[0].at[wid].at[pl.ds(0, IBLK)], si.at[0], isem)
    hd = pltpu.async_copy(e_hbm.at[1].at[wid].at[pl.ds(0, IBLK)], di.at[0], isem)
    _zero_vmem_2d(zrow, ZCH, H)
    hs.wait()
    hd.wait()
    pltpu.async_copy(y_hbm.at[si.at[0].at[0]], rows.at[0], gsem0)
    pltpu.async_copy(y_hbm.at[si.at[0].at[1]], rows.at[1], gsem1)
    for k in range(RPT // ZCH):
        pltpu.sync_copy(zrow, acc_sh.at[pl.ds(s * RPT + k * ZCH, ZCH)])
    plsc.subcore_barrier()

    for b in range(NBLK):
        p = b % 2
        q = (b + 1) % 2
        sb = si.at[p]
        db = di.at[p]
        if b + 1 < NBLK:
            nxt = pl.ds((b + 1) * IBLK, IBLK)
            hs = pltpu.async_copy(e_hbm.at[0].at[wid].at[nxt], si.at[q], isem)
            hd = pltpu.async_copy(e_hbm.at[1].at[wid].at[nxt], di.at[q], isem)
        if b > 0:
            pltpu.async_copy(y_hbm.at[sb.at[0]], rows.at[0], gsem0)
            pltpu.async_copy(y_hbm.at[sb.at[1]], rows.at[1], gsem1)

        def step(g, _, sb=sb, db=db):
            c0 = 2 * g
            pltpu.make_async_copy(y_hbm.at[sb.at[c0]], rows.at[0], gsem0).wait()
            pltpu.sync_copy(rows.at[0], acc_sh.at[db.at[c0]], add=True)
            pltpu.async_copy(y_hbm.at[sb.at[c0 + 2]], rows.at[0], gsem0)
            c1 = c0 + 1
            pltpu.make_async_copy(y_hbm.at[sb.at[c1]], rows.at[1], gsem1).wait()
            pltpu.sync_copy(rows.at[1], acc_sh.at[db.at[c1]], add=True)
            pltpu.async_copy(y_hbm.at[sb.at[c1 + 2]], rows.at[1], gsem1)
            return 0

        lax.fori_loop(0, IBLK // 2 - 1, step, 0, unroll=False)
        k0 = IBLK - 2
        pltpu.make_async_copy(y_hbm.at[sb.at[k0]], rows.at[0], gsem0).wait()
        pltpu.sync_copy(rows.at[0], acc_sh.at[db.at[k0]], add=True)
        pltpu.make_async_copy(y_hbm.at[sb.at[k0 + 1]], rows.at[1], gsem1).wait()
        pltpu.sync_copy(rows.at[1], acc_sh.at[db.at[k0 + 1]], add=True)
        if b + 1 < NBLK:
            hs.wait()
            hd.wait()

    plsc.subcore_barrier()
    for k in range(RPT // ZCH):
        r0 = s * RPT + k * ZCH
        pltpu.sync_copy(
            acc_sh.at[pl.ds(r0, ZCH)], out_hbm.at[c].at[pl.ds(r0, ZCH)]
        )


def _elu(v):
    return jnp.where(v > 0, v, jnp.exp(jnp.minimum(v, 0.0)) - 1.0)


def _tc_matmul1(x, W1):
    def body(x_ref, w_ref, xw_ref):
        xw_ref[...] = jnp.dot(
            x_ref[...], w_ref[...], preferred_element_type=jnp.float32
        )

    return pl.pallas_call(
        body,
        out_shape=jax.ShapeDtypeStruct((N, H), jnp.float32),
    )(x, W1)


def _tc_scale1(xw, cnt):
    def body(xw_ref, cnt_ref, y_ref, dinv_ref):
        cnt = cnt_ref[...]
        deg = cnt[0, :N] + cnt[1, :N] + 1.0
        dinv = jnp.reshape(lax.rsqrt(deg), (N, 1))
        y_ref[...] = xw_ref[...] * dinv
        dinv_ref[...] = dinv

    return pl.pallas_call(
        body,
        out_shape=(
            jax.ShapeDtypeStruct((N, H), jnp.float32),
            jax.ShapeDtypeStruct((N, 1), jnp.float32),
        ),
    )(xw, cnt)


def _tc_dense2(acc, y1, dinv, b1, W2):
    def body(acc_ref, y1_ref, dinv_ref, b1_ref, w_ref, y2_ref):
        dinv = dinv_ref[...]
        agg = acc_ref[0, :N, :] + acc_ref[1, :N, :] + y1_ref[...]
        h = _elu(dinv * agg + b1_ref[...])
        y2_ref[...] = (
            jnp.dot(h, w_ref[...], preferred_element_type=jnp.float32) * dinv
        )

    return pl.pallas_call(
        body,
        out_shape=jax.ShapeDtypeStruct((N, H), jnp.float32),
    )(acc, y1, dinv, b1, W2)


def _tc_dense3(acc, y2, dinv, b2, Wl, bl):
    def body(acc_ref, y2_ref, dinv_ref, b2_ref, wl_ref, bl_ref, out_ref):
        dinv = dinv_ref[...]
        agg = acc_ref[0, :N, :] + acc_ref[1, :N, :] + y2_ref[...]
        h = _elu(dinv * agg + b2_ref[...])
        logits_t = lax.dot_general(
            wl_ref[...], h,
            dimension_numbers=(((0,), (1,)), ((), ())),
            preferred_element_type=jnp.float32,
        ) + bl_ref[...]
        m = jnp.max(logits_t, axis=0, keepdims=True)
        lse = m + jnp.log(
            jnp.sum(jnp.exp(logits_t - m), axis=0, keepdims=True)
        )
        out_ref[...] = logits_t - lse

    return pl.pallas_call(
        body,
        out_shape=jax.ShapeDtypeStruct((C, N), jnp.float32),
    )(acc, y2, dinv, b2, Wl, bl)


def kernel(x, edge_index, W1, b1, W2, b2, Wl, bl):
    e4 = jnp.reshape(edge_index.astype(jnp.int32), (2, NW, NCHUNKS, CHUNK))

    cnt = _sc_count(e4)
    xw1 = _tc_matmul1(x, W1)
    y1, dinv = _tc_scale1(xw1, cnt)
    acc1 = _sc_scatter(y1, e4)
    y2 = _tc_dense2(acc1, y1, dinv, jnp.reshape(b1, (1, H)), W2)
    acc2 = _sc_scatter(y2, e4)
    out_t = _tc_dense3(
        acc2, y2, dinv, jnp.reshape(b2, (1, H)), Wl, jnp.reshape(bl, (C, 1))
    )
    return jnp.transpose(out_t)
